# baseline (speedup 1.0000x reference)
import functools
import math

import jax
import jax.numpy as jnp
from jax import lax
from jax.experimental import pallas as pl
from jax.experimental.pallas import tpu as pltpu
from jax.experimental.pallas import tpu_sc as plsc

B, N, D = 8, 1024, 256
DP = D // 4
TOP_K = 8
SIM_THRESHOLD = 0.7
NEG = -1e30

NC, NS, L = 2, 16, 16
NW = NC * NS
ROWS_PER_W = (B * N) // NW
CHUNK = 16
NCHUNK = ROWS_PER_W // CHUNK
GATHER = CHUNK * TOP_K


def _topk_body(sinv_ref, act_ref, wq_ref, bq_ref, wk_ref, bk_ref,
               w8_ref, idx_ref):
    a = act_ref[0]
    q = jnp.dot(a, wq_ref[...], preferred_element_type=jnp.float32) + bq_ref[...]
    k = jnp.dot(a, wk_ref[...], preferred_element_type=jnp.float32) + bk_ref[...]
    s = lax.dot_general(q, k, (((1,), (1,)), ((), ())),
                        preferred_element_type=jnp.float32)
    s = s * sinv_ref[0, 0]
    col = lax.broadcasted_iota(jnp.int32, (N, N), 1).astype(jnp.float32)
    vals, idxs = [], []
    for t in range(TOP_K):
        m = jnp.max(s, axis=1, keepdims=True)
        is_m = s == m
        am = jnp.min(jnp.where(is_m, col, 2048.0), axis=1, keepdims=True)
        vals.append(m)
        idxs.append(am)
        if t < TOP_K - 1:
            s = jnp.where(col == am, NEG, s)
    tv = jnp.concatenate(vals, axis=1)
    ti = jnp.concatenate(idxs, axis=1)
    e = jnp.exp(tv - tv[:, :1])
    w = e / jnp.sum(e, axis=1, keepdims=True)
    b = pl.program_id(0)
    w8_ref[0] = w
    idx_ref[0] = ti.astype(jnp.int32) + b * N


def _topk_call(sinv, actions, wq, bq, wk, bk):
    return pl.pallas_call(
        _topk_body,
        grid=(B,),
        in_specs=[
            pl.BlockSpec(memory_space=pltpu.SMEM),
            pl.BlockSpec((1, N, D), lambda b: (b, 0, 0)),
            pl.BlockSpec((D, DP), lambda b: (0, 0)),
            pl.BlockSpec((1, DP), lambda b: (0, 0)),
            pl.BlockSpec((D, DP), lambda b: (0, 0)),
            pl.BlockSpec((1, DP), lambda b: (0, 0)),
        ],
        out_specs=[
            pl.BlockSpec((1, N, TOP_K), lambda b: (b, 0, 0)),
            pl.BlockSpec((1, N, TOP_K), lambda b: (b, 0, 0)),
        ],
        out_shape=[
            jax.ShapeDtypeStruct((B, N, TOP_K), jnp.float32),
            jax.ShapeDtypeStruct((B, N, TOP_K), jnp.int32),
        ],
    )(sinv, actions, wq, bq, wk, bk)


def _wsim_body(t_ref, act_ref, wc_ref, bc_ref, wsim_ref):
    a = act_ref[0]
    p = jnp.dot(a, wc_ref[...], preferred_element_type=jnp.float32) + bc_ref[...]
    nrm = jnp.sqrt(jnp.sum(p * p, axis=1, keepdims=True))
    pn = (p / jnp.maximum(nrm, 1e-12)).astype(jnp.bfloat16)
    sim = lax.dot_general(pn, pn, (((1,), (1,)), ((), ())),
                          preferred_element_type=jnp.float32)
    wsim = jax.nn.sigmoid((sim - SIM_THRESHOLD) * t_ref[0, 0])
    wsim = wsim / (jnp.sum(wsim, axis=1, keepdims=True) + 1e-8)
    wsim_ref[0] = wsim.astype(jnp.bfloat16)


def _wsim_call(t10, actions, wc, bc):
    return pl.pallas_call(
        _wsim_body,
        grid=(B,),
        in_specs=[
            pl.BlockSpec(memory_space=pltpu.SMEM),
            pl.BlockSpec((1, N, D), lambda b: (b, 0, 0)),
            pl.BlockSpec((D, DP), lambda b: (0, 0)),
            pl.BlockSpec((1, DP), lambda b: (0, 0)),
        ],
        out_specs=pl.BlockSpec((1, N, N), lambda b: (b, 0, 0)),
        out_shape=jax.ShapeDtypeStruct((B, N, N), jnp.bfloat16),
    )(t10, actions, wc, bc)


NCH_TOTAL = (B * N) // CHUNK


def _sc_gather_body(states_hbm, idx_hbm, w_hbm, out_hbm,
                    idx_v, w_v, rows_v, out_v, sem_g0, sem_g1, sem_o0, sem_o1):
    wid = lax.axis_index("s") * NC + lax.axis_index("c")
    base_row = wid * ROWS_PER_W
    sem_g = (sem_g0, sem_g1)
    sem_o = (sem_o0, sem_o1)

    pltpu.sync_copy(idx_hbm.at[pl.ds(base_row * TOP_K, ROWS_PER_W * TOP_K)], idx_v)
    pltpu.sync_copy(w_hbm.at[pl.ds(base_row * TOP_K, ROWS_PER_W * TOP_K)], w_v)

    def gather_start(c, slot):
        idx_ref = idx_v.at[pl.ds(c * GATHER, GATHER)]
        pltpu.async_copy(states_hbm.at[idx_ref], rows_v.at[slot], sem_g[slot])

    def gather_wait(c, slot):
        idx_ref = idx_v.at[pl.ds(c * GATHER, GATHER)]
        pltpu.make_async_copy(states_hbm.at[idx_ref], rows_v.at[slot],
                              sem_g[slot]).wait()

    def out_start(c, slot):
        pltpu.async_copy(out_v.at[slot],
                         out_hbm.at[pl.ds(base_row + c * CHUNK, CHUNK)],
                         sem_o[slot])

    def out_wait(slot):
        pltpu.make_async_copy(out_v.at[slot],
                              out_hbm.at[pl.ds(base_row, CHUNK)],
                              sem_o[slot]).wait()

    def compute(c, slot):
        wbase = c * GATHER

        @plsc.parallel_loop(0, CHUNK, 1, unroll=2)
        def _row(i):
            rb = i * TOP_K
            acc = [None] * (D // 16)
            for j in range(TOP_K):
                wv = plsc.load_gather(
                    w_v, [jnp.full((16,), wbase + rb + j, jnp.int32)])
                for d in range(D // 16):
                    r = rows_v[slot, rb + j, pl.ds(d * 16, 16)]
                    acc[d] = wv * r if j == 0 else acc[d] + wv * r
            for d in range(D // 16):
                out_v[slot, i, pl.ds(d * 16, 16)] = acc[d]

    gather_start(0, 0)

    def pair_body(g2, carry):
        c0 = g2 * 2
        gather_start(c0 + 1, 1)
        gather_wait(c0, 0)

        @pl.when(g2 > 0)
        def _():
            out_wait(0)

        compute(c0, 0)
        out_start(c0, 0)
        gather_start(c0 + 2, 0)
        gather_wait(c0 + 1, 1)

        @pl.when(g2 > 0)
        def _():
            out_wait(1)

        compute(c0 + 1, 1)
        out_start(c0 + 1, 1)
        return carry

    lax.fori_loop(0, NCHUNK // 2 - 1, pair_body, 0)

    gather_start(NCHUNK - 1, 1)
    gather_wait(NCHUNK - 2, 0)
    out_wait(0)
    compute(NCHUNK - 2, 0)
    out_start(NCHUNK - 2, 0)
    gather_wait(NCHUNK - 1, 1)
    out_wait(1)
    compute(NCHUNK - 1, 1)
    out_start(NCHUNK - 1, 1)
    out_wait(0)
    out_wait(1)


def _sc_gather(states_flat, idx_flat, w_flat):
    mesh = plsc.VectorSubcoreMesh(core_axis_name="c", subcore_axis_name="s")
    f = pl.kernel(
        _sc_gather_body,
        out_type=jax.ShapeDtypeStruct((B * N, D), jnp.float32),
        mesh=mesh,
        scratch_types=[
            pltpu.VMEM((ROWS_PER_W * TOP_K,), jnp.int32),
            pltpu.VMEM((ROWS_PER_W * TOP_K,), jnp.float32),
            pltpu.VMEM((2, GATHER, D), jnp.float32),
            pltpu.VMEM((2, CHUNK, D), jnp.float32),
            pltpu.SemaphoreType.DMA,
            pltpu.SemaphoreType.DMA,
            pltpu.SemaphoreType.DMA,
            pltpu.SemaphoreType.DMA,
        ],
        compiler_params=pltpu.CompilerParams(needs_layout_passes=False),
    )
    return f(states_flat, idx_flat, w_flat)


def _combine_body(wsim_ref, inc_ref, out_ref):
    inc = inc_ref[0]
    comb = jnp.dot(wsim_ref[0], inc.astype(jnp.bfloat16),
                   preferred_element_type=jnp.float32)
    out_ref[0] = 0.8 * inc + 0.2 * comb


def _combine_call(wsim, incoming):
    return pl.pallas_call(
        _combine_body,
        grid=(B,),
        in_specs=[
            pl.BlockSpec((1, N, N), lambda b: (b, 0, 0)),
            pl.BlockSpec((1, N, D), lambda b: (b, 0, 0)),
        ],
        out_specs=pl.BlockSpec((1, N, D), lambda b: (b, 0, 0)),
        out_shape=jax.ShapeDtypeStruct((B, N, D), jnp.float32),
    )(wsim, incoming)


def kernel(agent_states, agent_actions, Wq, bq, Wk, bk, log_temperature, Wc, bc, temperature):
    temp = jnp.clip(jnp.exp(log_temperature), 0.1, 10.0)
    sinv = jnp.reshape(1.0 / (math.sqrt(DP) * temp), (1, 1)).astype(jnp.float32)
    t10 = jnp.reshape(temperature, (1, 1)).astype(jnp.float32)
    w8, idx8 = _topk_call(sinv, agent_actions, Wq,
                          bq.reshape(1, DP), Wk, bk.reshape(1, DP))
    states_flat = agent_states.reshape(B * N, D)
    inc_flat = _sc_gather(states_flat, idx8.reshape(-1), w8.reshape(-1))
    wsim = _wsim_call(t10, agent_actions, Wc, bc.reshape(1, DP))
    incoming = inc_flat.reshape(B, N, D)
    return _combine_call(wsim, incoming)


# speedup vs baseline: 1.0371x; 1.0371x over previous
import functools
import math

import jax
import jax.numpy as jnp
from jax import lax
from jax.experimental import pallas as pl
from jax.experimental.pallas import tpu as pltpu
from jax.experimental.pallas import tpu_sc as plsc

B, N, D = 8, 1024, 256
DP = D // 4
TOP_K = 8
SIM_THRESHOLD = 0.7
NEG = -1e30

NC, NS, L = 2, 16, 16
NW = NC * NS
ROWS_PER_W = (B * N) // NW
CHUNK = 16
NCHUNK = ROWS_PER_W // CHUNK
GATHER = CHUNK * TOP_K


def _topk_body(sinv_ref, act_ref, wq_ref, bq_ref, wk_ref, bk_ref,
               w8_ref, idx_ref):
    a = act_ref[0]
    q = jnp.dot(a, wq_ref[...], preferred_element_type=jnp.float32) + bq_ref[...]
    k = jnp.dot(a, wk_ref[...], preferred_element_type=jnp.float32) + bk_ref[...]
    s = lax.dot_general(q, k, (((1,), (1,)), ((), ())),
                        preferred_element_type=jnp.float32)
    s = s * sinv_ref[0, 0]
    col = lax.broadcasted_iota(jnp.int32, (N, N), 1).astype(jnp.float32)
    vals, idxs = [], []
    for t in range(TOP_K):
        m = jnp.max(s, axis=1, keepdims=True)
        is_m = s == m
        am = jnp.min(jnp.where(is_m, col, 2048.0), axis=1, keepdims=True)
        vals.append(m)
        idxs.append(am)
        if t < TOP_K - 1:
            s = jnp.where(col == am, NEG, s)
    tv = jnp.concatenate(vals, axis=1)
    ti = jnp.concatenate(idxs, axis=1)
    e = jnp.exp(tv - tv[:, :1])
    w = e / jnp.sum(e, axis=1, keepdims=True)
    b = pl.program_id(0)
    w8_ref[0] = w
    idx_ref[0] = ti.astype(jnp.int32) + b * N


def _topk_call(sinv, actions, wq, bq, wk, bk):
    return pl.pallas_call(
        _topk_body,
        grid=(B,),
        in_specs=[
            pl.BlockSpec(memory_space=pltpu.SMEM),
            pl.BlockSpec((1, N, D), lambda b: (b, 0, 0)),
            pl.BlockSpec((D, DP), lambda b: (0, 0)),
            pl.BlockSpec((1, DP), lambda b: (0, 0)),
            pl.BlockSpec((D, DP), lambda b: (0, 0)),
            pl.BlockSpec((1, DP), lambda b: (0, 0)),
        ],
        out_specs=[
            pl.BlockSpec((1, N, TOP_K), lambda b: (b, 0, 0)),
            pl.BlockSpec((1, N, TOP_K), lambda b: (b, 0, 0)),
        ],
        out_shape=[
            jax.ShapeDtypeStruct((B, N, TOP_K), jnp.float32),
            jax.ShapeDtypeStruct((B, N, TOP_K), jnp.int32),
        ],
    )(sinv, actions, wq, bq, wk, bk)


def _wsim_body(t_ref, act_ref, wc_ref, bc_ref, wsim_ref):
    a = act_ref[0]
    p = jnp.dot(a, wc_ref[...], preferred_element_type=jnp.float32) + bc_ref[...]
    nrm = jnp.sqrt(jnp.sum(p * p, axis=1, keepdims=True))
    pn = (p / jnp.maximum(nrm, 1e-12)).astype(jnp.bfloat16)
    sim = lax.dot_general(pn, pn, (((1,), (1,)), ((), ())),
                          preferred_element_type=jnp.float32)
    wsim = jax.nn.sigmoid((sim - SIM_THRESHOLD) * t_ref[0, 0])
    wsim = wsim / (jnp.sum(wsim, axis=1, keepdims=True) + 1e-8)
    wsim_ref[0] = wsim.astype(jnp.bfloat16)


def _wsim_call(t10, actions, wc, bc):
    return pl.pallas_call(
        _wsim_body,
        grid=(B,),
        in_specs=[
            pl.BlockSpec(memory_space=pltpu.SMEM),
            pl.BlockSpec((1, N, D), lambda b: (b, 0, 0)),
            pl.BlockSpec((D, DP), lambda b: (0, 0)),
            pl.BlockSpec((1, DP), lambda b: (0, 0)),
        ],
        out_specs=pl.BlockSpec((1, N, N), lambda b: (b, 0, 0)),
        out_shape=jax.ShapeDtypeStruct((B, N, N), jnp.bfloat16),
    )(t10, actions, wc, bc)


NCH_TOTAL = (B * N) // CHUNK


def _sc_gather_body(states_hbm, idx_hbm, w_hbm, out_hbm,
                    idx_v, w_v, rows_v, out_v, sem_g0, sem_g1, sem_o0, sem_o1):
    wid = lax.axis_index("s") * NC + lax.axis_index("c")
    base_row = wid * ROWS_PER_W
    sem_g = (sem_g0, sem_g1)
    sem_o = (sem_o0, sem_o1)

    pltpu.sync_copy(idx_hbm.at[pl.ds(base_row * TOP_K, ROWS_PER_W * TOP_K)], idx_v)
    pltpu.sync_copy(w_hbm.at[pl.ds(base_row * TOP_K, ROWS_PER_W * TOP_K)], w_v)

    def gather_start(c, slot):
        idx_ref = idx_v.at[pl.ds(c * GATHER, GATHER)]
        pltpu.async_copy(states_hbm.at[idx_ref], rows_v.at[slot], sem_g[slot])

    def gather_wait(c, slot):
        idx_ref = idx_v.at[pl.ds(c * GATHER, GATHER)]
        pltpu.make_async_copy(states_hbm.at[idx_ref], rows_v.at[slot],
                              sem_g[slot]).wait()

    def out_start(c, slot):
        pltpu.async_copy(out_v.at[slot],
                         out_hbm.at[pl.ds(base_row + c * CHUNK, CHUNK)],
                         sem_o[slot])

    def out_wait(slot):
        pltpu.make_async_copy(out_v.at[slot],
                              out_hbm.at[pl.ds(base_row, CHUNK)],
                              sem_o[slot]).wait()

    def compute(c, slot):
        wbase = c * GATHER

        def row_body(i2, carry):
            for u in range(2):
                i = i2 * 2 + u
                rb = i * TOP_K
                acc = [None] * (D // 16)
                for j in range(TOP_K):
                    wv = plsc.load_gather(
                        w_v, [jnp.full((16,), wbase + rb + j, jnp.int32)])
                    for d in range(D // 16):
                        r = rows_v[slot, rb + j, pl.ds(d * 16, 16)]
                        acc[d] = wv * r if j == 0 else acc[d] + wv * r
                for d in range(D // 16):
                    out_v[slot, i, pl.ds(d * 16, 16)] = acc[d]
            return carry

        lax.fori_loop(0, CHUNK // 2, row_body, 0)

    gather_start(0, 0)

    def pair_body(g2, carry):
        c0 = g2 * 2
        gather_start(c0 + 1, 1)
        gather_wait(c0, 0)

        @pl.when(g2 > 0)
        def _():
            out_wait(0)

        compute(c0, 0)
        out_start(c0, 0)
        gather_start(c0 + 2, 0)
        gather_wait(c0 + 1, 1)

        @pl.when(g2 > 0)
        def _():
            out_wait(1)

        compute(c0 + 1, 1)
        out_start(c0 + 1, 1)
        return carry

    lax.fori_loop(0, NCHUNK // 2 - 1, pair_body, 0)

    gather_start(NCHUNK - 1, 1)
    gather_wait(NCHUNK - 2, 0)
    out_wait(0)
    compute(NCHUNK - 2, 0)
    out_start(NCHUNK - 2, 0)
    gather_wait(NCHUNK - 1, 1)
    out_wait(1)
    compute(NCHUNK - 1, 1)
    out_start(NCHUNK - 1, 1)
    out_wait(0)
    out_wait(1)


def _sc_gather(states_flat, idx_flat, w_flat):
    mesh = plsc.VectorSubcoreMesh(core_axis_name="c", subcore_axis_name="s")
    f = pl.kernel(
        _sc_gather_body,
        out_type=jax.ShapeDtypeStruct((B * N, D), jnp.float32),
        mesh=mesh,
        scratch_types=[
            pltpu.VMEM((ROWS_PER_W * TOP_K,), jnp.int32),
            pltpu.VMEM((ROWS_PER_W * TOP_K,), jnp.float32),
            pltpu.VMEM((2, GATHER, D), jnp.float32),
            pltpu.VMEM((2, CHUNK, D), jnp.float32),
            pltpu.SemaphoreType.DMA,
            pltpu.SemaphoreType.DMA,
            pltpu.SemaphoreType.DMA,
            pltpu.SemaphoreType.DMA,
        ],
        compiler_params=pltpu.CompilerParams(needs_layout_passes=False),
    )
    return f(states_flat, idx_flat, w_flat)


def _combine_body(wsim_ref, inc_ref, out_ref):
    inc = inc_ref[0]
    comb = jnp.dot(wsim_ref[0], inc.astype(jnp.bfloat16),
                   preferred_element_type=jnp.float32)
    out_ref[0] = 0.8 * inc + 0.2 * comb


def _combine_call(wsim, incoming):
    return pl.pallas_call(
        _combine_body,
        grid=(B,),
        in_specs=[
            pl.BlockSpec((1, N, N), lambda b: (b, 0, 0)),
            pl.BlockSpec((1, N, D), lambda b: (b, 0, 0)),
        ],
        out_specs=pl.BlockSpec((1, N, D), lambda b: (b, 0, 0)),
        out_shape=jax.ShapeDtypeStruct((B, N, D), jnp.float32),
    )(wsim, incoming)


def kernel(agent_states, agent_actions, Wq, bq, Wk, bk, log_temperature, Wc, bc, temperature):
    temp = jnp.clip(jnp.exp(log_temperature), 0.1, 10.0)
    sinv = jnp.reshape(1.0 / (math.sqrt(DP) * temp), (1, 1)).astype(jnp.float32)
    t10 = jnp.reshape(temperature, (1, 1)).astype(jnp.float32)
    w8, idx8 = _topk_call(sinv, agent_actions, Wq,
                          bq.reshape(1, DP), Wk, bk.reshape(1, DP))
    states_flat = agent_states.reshape(B * N, D)
    inc_flat = _sc_gather(states_flat, idx8.reshape(-1), w8.reshape(-1))
    wsim = _wsim_call(t10, agent_actions, Wc, bc.reshape(1, DP))
    incoming = inc_flat.reshape(B, N, D)
    return _combine_call(wsim, incoming)
